# baseline (speedup 1.0000x reference)
import functools

import jax
import jax.numpy as jnp
from jax import lax
from jax.experimental import pallas as pl
from jax.experimental.pallas import tpu as pltpu
from jax.experimental.pallas import tpu_sc as plsc

VOCAB = 100000
DIM = 64
BATCH = 4096
LENGTH = 200
N = BATCH * LENGTH
NWIDE = N // 2

NC = 2
NS = 16
NW = NC * NS
WR = NWIDE // NW
CW = 256
STEPS = WR // CW
NBUF = 2
GROUPS = STEPS // NBUF

_mesh = plsc.VectorSubcoreMesh(core_axis_name="c", subcore_axis_name="s")


@functools.partial(
    pl.kernel,
    mesh=_mesh,
    out_type=jax.ShapeDtypeStruct((NWIDE, 2 * DIM), jnp.float32),
    scratch_types=(
        [pltpu.VMEM((WR,), jnp.int32) for _ in range(2)]
        + [pltpu.VMEM((CW, DIM), jnp.float32) for _ in range(2 * NBUF)]
        + [pltpu.SemaphoreType.DMA for _ in range(2 * NBUF)]
    ),
    compiler_params=pltpu.CompilerParams(use_tc_tiling_on_sc=False),
)
def _gather_kernel(ev_hbm, od_hbm, table_hbm, out_hbm, ev_all, od_all,
                   *bufs_and_sems):
    ev_rows = bufs_and_sems[0:NBUF]
    od_rows = bufs_and_sems[NBUF:2 * NBUF]
    g_sems = bufs_and_sems[2 * NBUF:3 * NBUF]
    o_sems = bufs_and_sems[3 * NBUF:]

    wid = lax.axis_index("s") * NC + lax.axis_index("c")
    base = wid * WR

    pltpu.sync_copy(ev_hbm.at[pl.ds(base, WR)], ev_all)
    pltpu.sync_copy(od_hbm.at[pl.ds(base, WR)], od_all)

    def gathers(b, chunk_i):
        lo = chunk_i * CW
        return (
            pltpu.make_async_copy(
                table_hbm.at[ev_all.at[pl.ds(lo, CW)]], ev_rows[b],
                g_sems[b]),
            pltpu.make_async_copy(
                table_hbm.at[od_all.at[pl.ds(lo, CW)]], od_rows[b],
                g_sems[b]),
        )

    def out_copies(b, chunk_i):
        lo = base + chunk_i * CW
        return (
            pltpu.make_async_copy(
                ev_rows[b], out_hbm.at[pl.ds(lo, CW), pl.ds(0, DIM)],
                o_sems[b]),
            pltpu.make_async_copy(
                od_rows[b], out_hbm.at[pl.ds(lo, CW), pl.ds(DIM, DIM)],
                o_sems[b]),
        )

    for b in range(NBUF):
        for c in gathers(b, b):
            c.start()

    def group(g, carry):
        for b in range(NBUF):
            i = g * NBUF + b
            for c in gathers(b, i):
                c.wait()
            for c in out_copies(b, i):
                c.start()
        for b in range(NBUF):
            i_next = (g + 1) * NBUF + b

            @pl.when(i_next < STEPS)
            def _():
                for c in out_copies(b, i_next - NBUF):
                    c.wait()
                for c in gathers(b, i_next):
                    c.start()

        return carry

    lax.fori_loop(0, GROUPS, group, 0)

    for b in range(NBUF):
        for c in out_copies(b, STEPS - NBUF + b):
            c.wait()


def kernel(wids, wordemb):
    flat = wids.reshape(-1).astype(jnp.int32)
    out = _gather_kernel(flat[0::2], flat[1::2], wordemb)
    return out.reshape(BATCH, LENGTH, DIM)


# speedup vs baseline: 1.2017x; 1.2017x over previous
import functools

import jax
import jax.numpy as jnp
from jax import lax
from jax.experimental import pallas as pl
from jax.experimental.pallas import tpu as pltpu
from jax.experimental.pallas import tpu_sc as plsc

VOCAB = 100000
DIM = 64
BATCH = 4096
LENGTH = 200
N = BATCH * LENGTH

NC = 2
NS = 16
NW = NC * NS
WR = N // NW
CW = 512
STEPS = WR // CW
NBUF = 2
GROUPS = STEPS // NBUF

_mesh = plsc.VectorSubcoreMesh(core_axis_name="c", subcore_axis_name="s")


@functools.partial(
    pl.kernel,
    mesh=_mesh,
    out_type=jax.ShapeDtypeStruct((N, DIM), jnp.float32),
    scratch_types=(
        [pltpu.VMEM((WR,), jnp.int32)]
        + [pltpu.VMEM((CW, DIM), jnp.float32) for _ in range(NBUF)]
        + [pltpu.SemaphoreType.DMA for _ in range(2 * NBUF)]
    ),
    compiler_params=pltpu.CompilerParams(use_tc_tiling_on_sc=False),
)
def _gather_kernel(idx_hbm, table_hbm, out_hbm, idx_all, *bufs_and_sems):
    rows = bufs_and_sems[0:NBUF]
    g_sems = bufs_and_sems[NBUF:2 * NBUF]
    o_sems = bufs_and_sems[2 * NBUF:]

    wid = lax.axis_index("s") * NC + lax.axis_index("c")
    base = wid * WR

    pltpu.sync_copy(idx_hbm.at[pl.ds(base, WR)], idx_all)

    def gather(b, chunk_i):
        return pltpu.make_async_copy(
            table_hbm.at[idx_all.at[pl.ds(chunk_i * CW, CW)]], rows[b],
            g_sems[b])

    def out_copy(b, chunk_i):
        return pltpu.make_async_copy(
            rows[b], out_hbm.at[pl.ds(base + chunk_i * CW, CW)], o_sems[b])

    for b in range(NBUF):
        gather(b, b).start()

    def group(g, carry):
        for b in range(NBUF):
            i = g * NBUF + b
            gather(b, i).wait()
            out_copy(b, i).start()
        for b in range(NBUF):
            i_next = (g + 1) * NBUF + b

            @pl.when(i_next < STEPS)
            def _():
                out_copy(b, i_next - NBUF).wait()
                gather(b, i_next).start()

        return carry

    lax.fori_loop(0, GROUPS, group, 0)

    for b in range(NBUF):
        out_copy(b, STEPS - NBUF + b).wait()


def kernel(wids, wordemb):
    flat = wids.reshape(-1).astype(jnp.int32)
    out = _gather_kernel(flat, wordemb)
    return out.reshape(BATCH, LENGTH, DIM)


# speedup vs baseline: 1.2133x; 1.0096x over previous
import functools

import jax
import jax.numpy as jnp
from jax import lax
from jax.experimental import pallas as pl
from jax.experimental.pallas import tpu as pltpu
from jax.experimental.pallas import tpu_sc as plsc

VOCAB = 100000
DIM = 64
BATCH = 4096
LENGTH = 200
N = BATCH * LENGTH

NC = 2
NS = 16
NW = NC * NS
WR = N // NW
CW = 256
STEPS = WR // CW
NBUF = 4
GROUPS = STEPS // NBUF

_mesh = plsc.VectorSubcoreMesh(core_axis_name="c", subcore_axis_name="s")


@functools.partial(
    pl.kernel,
    mesh=_mesh,
    out_type=jax.ShapeDtypeStruct((N, DIM), jnp.float32),
    scratch_types=(
        [pltpu.VMEM((WR,), jnp.int32)]
        + [pltpu.VMEM((CW, DIM), jnp.float32) for _ in range(NBUF)]
        + [pltpu.SemaphoreType.DMA for _ in range(2 * NBUF)]
    ),
    compiler_params=pltpu.CompilerParams(use_tc_tiling_on_sc=False),
)
def _gather_kernel(idx_hbm, table_hbm, out_hbm, idx_all, *bufs_and_sems):
    rows = bufs_and_sems[0:NBUF]
    g_sems = bufs_and_sems[NBUF:2 * NBUF]
    o_sems = bufs_and_sems[2 * NBUF:]

    wid = lax.axis_index("s") * NC + lax.axis_index("c")
    base = wid * WR

    pltpu.sync_copy(idx_hbm.at[pl.ds(base, WR)], idx_all)

    def gather(b, chunk_i):
        return pltpu.make_async_copy(
            table_hbm.at[idx_all.at[pl.ds(chunk_i * CW, CW)]], rows[b],
            g_sems[b])

    def out_copy(b, chunk_i):
        return pltpu.make_async_copy(
            rows[b], out_hbm.at[pl.ds(base + chunk_i * CW, CW)], o_sems[b])

    for b in range(NBUF):
        gather(b, b).start()

    def group(g, carry):
        for b in range(NBUF):
            i = g * NBUF + b
            gather(b, i).wait()
            out_copy(b, i).start()
        for b in range(NBUF):
            i_next = (g + 1) * NBUF + b

            @pl.when(i_next < STEPS)
            def _():
                out_copy(b, i_next - NBUF).wait()
                gather(b, i_next).start()

        return carry

    lax.fori_loop(0, GROUPS, group, 0)

    for b in range(NBUF):
        out_copy(b, STEPS - NBUF + b).wait()


def kernel(wids, wordemb):
    flat = wids.reshape(-1).astype(jnp.int32)
    out = _gather_kernel(flat, wordemb)
    return out.reshape(BATCH, LENGTH, DIM)
